# baseline (speedup 1.0000x reference)
import functools

import jax
import jax.numpy as jnp
from jax import lax
from jax.experimental import pallas as pl
from jax.experimental.pallas import tpu as pltpu
from jax.experimental.pallas import tpu_sc as plsc

VOCAB = 1000000
SEQ = 200
BATCH = 4096
EMB = 64
OUT = 16
LANES = 16
NCORES = 2
NSUB = 16
NW = NCORES * NSUB
BPW = BATCH // NW
NBUF = 6
UNROLL = 16

SPLIT = 1 << 19
TBLK = 4096
TGRID = SPLIT // TBLK


def _tc_transpose(tt):
    def body(a_ref, b_ref, o_ref):
        o_ref[:, 0:EMB] = a_ref[...].T
        o_ref[:, EMB:2 * EMB] = b_ref[...].T

    return pl.pallas_call(
        body,
        out_shape=jax.ShapeDtypeStruct((SPLIT, 2 * EMB), jnp.float32),
        grid=(TGRID,),
        in_specs=[
            pl.BlockSpec((EMB, TBLK), lambda i: (0, i)),
            pl.BlockSpec((EMB, TBLK),
                         lambda i: (0, jnp.minimum(TGRID + i, VOCAB // TBLK))),
        ],
        out_specs=pl.BlockSpec((TBLK, 2 * EMB), lambda i: (i, 0)),
    )(tt, tt)


def _sc_pool(x, table_lin):
    mesh = plsc.VectorSubcoreMesh(core_axis_name="c", subcore_axis_name="s")

    @functools.partial(
        pl.kernel,
        out_type=jax.ShapeDtypeStruct((BATCH, EMB), jnp.float32),
        mesh=mesh,
        scratch_types=[
            pltpu.VMEM((SEQ, BPW), jnp.int32),
            [pltpu.VMEM((BPW, EMB), jnp.float32) for _ in range(NBUF)],
            pltpu.VMEM((BPW, EMB), jnp.float32),
            [pltpu.SemaphoreType.DMA for _ in range(NBUF)],
        ],
        compiler_params=pltpu.CompilerParams(use_tc_tiling_on_sc=False),
    )
    def pool_kernel(x_hbm, tab_hbm, out_hbm, idx_v, rows, acc_v, sems):
        cid = lax.axis_index("c")
        sid = lax.axis_index("s")
        wid = sid * NCORES + cid
        base = wid * BPW

        pltpu.sync_copy(x_hbm.at[:, pl.ds(base, BPW)], idx_v)

        def remap_body(k, _):
            sl = pl.ds(k * LANES, LANES)
            i0 = idx_v[k // (BPW // LANES), pl.ds((k % (BPW // LANES)) * LANES, LANES)]
            h = lax.shift_right_logical(i0, 19)
            idx_v[k // (BPW // LANES), pl.ds((k % (BPW // LANES)) * LANES, LANES)] = (
                i0 * 2 - h * (2 * SPLIT - 1))
            return 0

        lax.fori_loop(0, SEQ * BPW // LANES, remap_body, 0)

        def gather(s, b):
            return pltpu.make_async_copy(tab_hbm.at[idx_v.at[s]], rows[b], sems[b])

        for b in range(NBUF):
            gather(b, b).start()

        zero = jnp.zeros((LANES,), jnp.float32)

        def zero_body(r, _):
            for c in range(EMB // LANES):
                acc_v[r, pl.ds(c * LANES, LANES)] = zero
            return 0

        lax.fori_loop(0, BPW, zero_body, 0)

        def accum(buf):
            def body(i, _):
                r0 = i * UNROLL
                for u in range(UNROLL):
                    for c in range(EMB // LANES):
                        sl = pl.ds(c * LANES, LANES)
                        plsc.addupdate(acc_v.at[r0 + u, sl], buf[r0 + u, sl])
                return 0

            lax.fori_loop(0, BPW // UNROLL, body, 0)

        def outer(g, _):
            for b in range(NBUF):
                s = g * NBUF + b

                @pl.when(s < SEQ)
                def _():
                    gather(s, b).wait()
                    accum(rows[b])
                    ns = s + NBUF

                    @pl.when(ns < SEQ)
                    def _():
                        gather(ns, b).start()

            return 0

        lax.fori_loop(0, (SEQ + NBUF - 1) // NBUF, outer, 0)

        pltpu.sync_copy(acc_v, out_hbm.at[pl.ds(base, BPW), :])

    return pool_kernel(x, table_lin)


def _tc_head(pooled, W, b2d):
    blk = 512

    def head_kernel(p_ref, w_ref, b_ref, o_ref):
        logits = jnp.dot(p_ref[...], w_ref[...],
                         preferred_element_type=jnp.float32) + b_ref[...]
        m = jnp.max(logits, axis=1, keepdims=True)
        z = logits - m
        lse = jnp.log(jnp.sum(jnp.exp(z), axis=1, keepdims=True))
        o_ref[...] = z - lse

    return pl.pallas_call(
        head_kernel,
        out_shape=jax.ShapeDtypeStruct((BATCH, OUT), jnp.float32),
        grid=(BATCH // blk,),
        in_specs=[
            pl.BlockSpec((blk, EMB), lambda i: (i, 0)),
            pl.BlockSpec((EMB, OUT), lambda i: (0, 0)),
            pl.BlockSpec((1, OUT), lambda i: (0, 0)),
        ],
        out_specs=pl.BlockSpec((blk, OUT), lambda i: (i, 0)),
    )(pooled, W, b2d)


def kernel(x, table, W, b):
    pairs = _tc_transpose(table.T)
    sums = _sc_pool(x, pairs.reshape(2 * SPLIT, EMB))
    return _tc_head(sums, W * jnp.float32(1.0 / SEQ), b.reshape(1, OUT))


# speedup vs baseline: 1.0503x; 1.0503x over previous
import functools

import jax
import jax.numpy as jnp
from jax import lax
from jax.experimental import pallas as pl
from jax.experimental.pallas import tpu as pltpu
from jax.experimental.pallas import tpu_sc as plsc

VOCAB = 1000000
SEQ = 200
BATCH = 4096
EMB = 64
OUT = 16
LANES = 16
NCORES = 2
NSUB = 16
NW = NCORES * NSUB
BPW = BATCH // NW
NBUF = 4
UNROLL = 8

SPLIT = 1 << 19
TBLK = 4096
TGRID = SPLIT // TBLK


def _tc_transpose(tt):
    def body(a_ref, b_ref, o_ref):
        o_ref[:, 0:EMB] = a_ref[...].T
        o_ref[:, EMB:2 * EMB] = b_ref[...].T

    return pl.pallas_call(
        body,
        out_shape=jax.ShapeDtypeStruct((SPLIT, 2 * EMB), jnp.float32),
        grid=(TGRID,),
        in_specs=[
            pl.BlockSpec((EMB, TBLK), lambda i: (0, i)),
            pl.BlockSpec((EMB, TBLK),
                         lambda i: (0, jnp.minimum(TGRID + i, VOCAB // TBLK))),
        ],
        out_specs=pl.BlockSpec((TBLK, 2 * EMB), lambda i: (i, 0)),
    )(tt, tt)


def _sc_pool(x, table_lin):
    mesh = plsc.VectorSubcoreMesh(core_axis_name="c", subcore_axis_name="s")

    @functools.partial(
        pl.kernel,
        out_type=jax.ShapeDtypeStruct((BATCH, EMB), jnp.float32),
        mesh=mesh,
        scratch_types=[
            pltpu.VMEM((SEQ, BPW), jnp.int32),
            [pltpu.VMEM((BPW, EMB), jnp.float32) for _ in range(NBUF)],
            pltpu.VMEM((BPW, EMB), jnp.float32),
            [pltpu.SemaphoreType.DMA for _ in range(NBUF)],
        ],
        compiler_params=pltpu.CompilerParams(use_tc_tiling_on_sc=False),
    )
    def pool_kernel(x_hbm, tab_hbm, out_hbm, idx_v, rows, acc_v, sems):
        cid = lax.axis_index("c")
        sid = lax.axis_index("s")
        wid = sid * NCORES + cid
        base = wid * BPW

        pltpu.sync_copy(x_hbm.at[:, pl.ds(base, BPW)], idx_v)

        def remap_body(k, _):
            sl = pl.ds(k * LANES, LANES)
            i0 = idx_v[k // (BPW // LANES), pl.ds((k % (BPW // LANES)) * LANES, LANES)]
            h = lax.shift_right_logical(i0, 19)
            idx_v[k // (BPW // LANES), pl.ds((k % (BPW // LANES)) * LANES, LANES)] = (
                i0 * 2 - h * (2 * SPLIT - 1))
            return 0

        lax.fori_loop(0, SEQ * BPW // LANES, remap_body, 0)

        def gather(s, b):
            return pltpu.make_async_copy(tab_hbm.at[idx_v.at[s]], rows[b], sems[b])

        for b in range(NBUF):
            gather(b, b).start()

        zero = jnp.zeros((LANES,), jnp.float32)

        def zero_body(r, _):
            for c in range(EMB // LANES):
                acc_v[r, pl.ds(c * LANES, LANES)] = zero
            return 0

        lax.fori_loop(0, BPW, zero_body, 0)

        def accum(buf):
            def body(i, _):
                r0 = i * UNROLL
                for u in range(UNROLL):
                    for c in range(EMB // LANES):
                        sl = pl.ds(c * LANES, LANES)
                        plsc.addupdate(acc_v.at[r0 + u, sl], buf[r0 + u, sl])
                return 0

            lax.fori_loop(0, BPW // UNROLL, body, 0)

        def outer(g, _):
            for b in range(NBUF):
                s = g * NBUF + b
                gather(s, b).wait()
                accum(rows[b])
                ns = s + NBUF

                @pl.when(ns < SEQ)
                def _():
                    gather(ns, b).start()

            return 0

        lax.fori_loop(0, SEQ // NBUF, outer, 0)

        pltpu.sync_copy(acc_v, out_hbm.at[pl.ds(base, BPW), :])

    return pool_kernel(x, table_lin)


def _tc_head(pooled, W, b2d):
    blk = 512

    def head_kernel(p_ref, w_ref, b_ref, o_ref):
        logits = jnp.dot(p_ref[...], w_ref[...],
                         preferred_element_type=jnp.float32) + b_ref[...]
        m = jnp.max(logits, axis=1, keepdims=True)
        z = logits - m
        lse = jnp.log(jnp.sum(jnp.exp(z), axis=1, keepdims=True))
        o_ref[...] = z - lse

    return pl.pallas_call(
        head_kernel,
        out_shape=jax.ShapeDtypeStruct((BATCH, OUT), jnp.float32),
        grid=(BATCH // blk,),
        in_specs=[
            pl.BlockSpec((blk, EMB), lambda i: (i, 0)),
            pl.BlockSpec((EMB, OUT), lambda i: (0, 0)),
            pl.BlockSpec((1, OUT), lambda i: (0, 0)),
        ],
        out_specs=pl.BlockSpec((blk, OUT), lambda i: (i, 0)),
    )(pooled, W, b2d)


def kernel(x, table, W, b):
    pairs = _tc_transpose(table.T)
    sums = _sc_pool(x, pairs.reshape(2 * SPLIT, EMB))
    return _tc_head(sums, W * jnp.float32(1.0 / SEQ), b.reshape(1, OUT))


# speedup vs baseline: 1.1423x; 1.0876x over previous
import functools

import jax
import jax.numpy as jnp
from jax import lax
from jax.experimental import pallas as pl
from jax.experimental.pallas import tpu as pltpu
from jax.experimental.pallas import tpu_sc as plsc

VOCAB = 1000000
SEQ = 200
BATCH = 4096
EMB = 64
OUT = 16
LANES = 16
NCORES = 2
NSUB = 16
NW = NCORES * NSUB
BPW = BATCH // NW
NBUF = 4
UNROLL = 8

SPLIT = 1 << 19
TBLK = 8192
TGRID = SPLIT // TBLK


def _tc_transpose(tt):
    def body(a_ref, b_ref, o_ref):
        o_ref[:, 0:EMB] = a_ref[...].T
        o_ref[:, EMB:2 * EMB] = b_ref[...].T

    return pl.pallas_call(
        body,
        out_shape=jax.ShapeDtypeStruct((SPLIT, 2 * EMB), jnp.float32),
        grid=(TGRID,),
        in_specs=[
            pl.BlockSpec((EMB, TBLK), lambda i: (0, i)),
            pl.BlockSpec((EMB, TBLK),
                         lambda i: (0, jnp.minimum(TGRID + i, VOCAB // TBLK))),
        ],
        out_specs=pl.BlockSpec((TBLK, 2 * EMB), lambda i: (i, 0)),
    )(tt, tt)


def _sc_pool(x, table_lin):
    mesh = plsc.VectorSubcoreMesh(core_axis_name="c", subcore_axis_name="s")

    @functools.partial(
        pl.kernel,
        out_type=jax.ShapeDtypeStruct((BATCH, EMB), jnp.float32),
        mesh=mesh,
        scratch_types=[
            pltpu.VMEM((SEQ, BPW), jnp.int32),
            [pltpu.VMEM((BPW, EMB), jnp.float32) for _ in range(NBUF)],
            pltpu.VMEM((BPW, EMB), jnp.float32),
            [pltpu.SemaphoreType.DMA for _ in range(NBUF)],
        ],
        compiler_params=pltpu.CompilerParams(use_tc_tiling_on_sc=False),
    )
    def pool_kernel(x_hbm, tab_hbm, out_hbm, idx_v, rows, acc_v, sems):
        cid = lax.axis_index("c")
        sid = lax.axis_index("s")
        wid = sid * NCORES + cid
        base = wid * BPW

        pltpu.sync_copy(x_hbm.at[:, pl.ds(base, BPW)], idx_v)

        def remap_body(k, _):
            sl = pl.ds(k * LANES, LANES)
            i0 = idx_v[k // (BPW // LANES), pl.ds((k % (BPW // LANES)) * LANES, LANES)]
            h = lax.shift_right_logical(i0, 19)
            idx_v[k // (BPW // LANES), pl.ds((k % (BPW // LANES)) * LANES, LANES)] = (
                i0 * 2 - h * (2 * SPLIT - 1))
            return 0

        lax.fori_loop(0, SEQ * BPW // LANES, remap_body, 0)

        def gather(s, b):
            return pltpu.make_async_copy(tab_hbm.at[idx_v.at[s]], rows[b], sems[b])

        for b in range(NBUF):
            gather(b, b).start()

        zero = jnp.zeros((LANES,), jnp.float32)

        def zero_body(r, _):
            for c in range(EMB // LANES):
                acc_v[r, pl.ds(c * LANES, LANES)] = zero
            return 0

        lax.fori_loop(0, BPW, zero_body, 0)

        def accum(buf):
            def body(i, _):
                r0 = i * UNROLL
                for u in range(UNROLL):
                    for c in range(EMB // LANES):
                        sl = pl.ds(c * LANES, LANES)
                        plsc.addupdate(acc_v.at[r0 + u, sl], buf[r0 + u, sl])
                return 0

            lax.fori_loop(0, BPW // UNROLL, body, 0)

        def outer(g, _):
            for b in range(NBUF):
                s = g * NBUF + b
                gather(s, b).wait()
                accum(rows[b])
                ns = s + NBUF

                @pl.when(ns < SEQ)
                def _():
                    gather(ns, b).start()

            return 0

        lax.fori_loop(0, SEQ // NBUF, outer, 0)

        pltpu.sync_copy(acc_v, out_hbm.at[pl.ds(base, BPW), :])

    return pool_kernel(x, table_lin)


def _tc_head(pooled, W, b2d):
    blk = 512

    def head_kernel(p_ref, w_ref, b_ref, o_ref):
        logits = jnp.dot(p_ref[...], w_ref[...],
                         preferred_element_type=jnp.float32) + b_ref[...]
        m = jnp.max(logits, axis=1, keepdims=True)
        z = logits - m
        lse = jnp.log(jnp.sum(jnp.exp(z), axis=1, keepdims=True))
        o_ref[...] = z - lse

    return pl.pallas_call(
        head_kernel,
        out_shape=jax.ShapeDtypeStruct((BATCH, OUT), jnp.float32),
        grid=(BATCH // blk,),
        in_specs=[
            pl.BlockSpec((blk, EMB), lambda i: (i, 0)),
            pl.BlockSpec((EMB, OUT), lambda i: (0, 0)),
            pl.BlockSpec((1, OUT), lambda i: (0, 0)),
        ],
        out_specs=pl.BlockSpec((blk, OUT), lambda i: (i, 0)),
    )(pooled, W, b2d)


def kernel(x, table, W, b):
    pairs = _tc_transpose(table.T)
    sums = _sc_pool(x, pairs.reshape(2 * SPLIT, EMB))
    return _tc_head(sums, W * jnp.float32(1.0 / SEQ), b.reshape(1, OUT))


# speedup vs baseline: 1.1787x; 1.0319x over previous
import functools

import jax
import jax.numpy as jnp
from jax import lax
from jax.experimental import pallas as pl
from jax.experimental.pallas import tpu as pltpu
from jax.experimental.pallas import tpu_sc as plsc

VOCAB = 1000000
SEQ = 200
BATCH = 4096
EMB = 64
OUT = 16
LANES = 16
NCORES = 2
NSUB = 16
NW = NCORES * NSUB
BPW = BATCH // NW
NBUF = 4
UNROLL = 8

SPLIT = 1 << 19
TBLK = 16384
TGRID = SPLIT // TBLK


def _tc_transpose(tt):
    def body(a_ref, b_ref, o_ref):
        o_ref[:, 0:EMB] = a_ref[...].T
        o_ref[:, EMB:2 * EMB] = b_ref[...].T

    return pl.pallas_call(
        body,
        out_shape=jax.ShapeDtypeStruct((SPLIT, 2 * EMB), jnp.float32),
        grid=(TGRID,),
        in_specs=[
            pl.BlockSpec((EMB, TBLK), lambda i: (0, i)),
            pl.BlockSpec((EMB, TBLK),
                         lambda i: (0, jnp.minimum(TGRID + i, VOCAB // TBLK))),
        ],
        out_specs=pl.BlockSpec((TBLK, 2 * EMB), lambda i: (i, 0)),
    )(tt, tt)


def _sc_pool(x, table_lin):
    mesh = plsc.VectorSubcoreMesh(core_axis_name="c", subcore_axis_name="s")

    @functools.partial(
        pl.kernel,
        out_type=jax.ShapeDtypeStruct((BATCH, EMB), jnp.float32),
        mesh=mesh,
        scratch_types=[
            pltpu.VMEM((SEQ, BPW), jnp.int32),
            [pltpu.VMEM((BPW, EMB), jnp.float32) for _ in range(NBUF)],
            pltpu.VMEM((BPW, EMB), jnp.float32),
            [pltpu.SemaphoreType.DMA for _ in range(NBUF)],
        ],
        compiler_params=pltpu.CompilerParams(use_tc_tiling_on_sc=False),
    )
    def pool_kernel(x_hbm, tab_hbm, out_hbm, idx_v, rows, acc_v, sems):
        cid = lax.axis_index("c")
        sid = lax.axis_index("s")
        wid = sid * NCORES + cid
        base = wid * BPW

        pltpu.sync_copy(x_hbm.at[:, pl.ds(base, BPW)], idx_v)

        def remap_body(k, _):
            sl = pl.ds(k * LANES, LANES)
            i0 = idx_v[k // (BPW // LANES), pl.ds((k % (BPW // LANES)) * LANES, LANES)]
            h = lax.shift_right_logical(i0, 19)
            idx_v[k // (BPW // LANES), pl.ds((k % (BPW // LANES)) * LANES, LANES)] = (
                i0 * 2 - h * (2 * SPLIT - 1))
            return 0

        lax.fori_loop(0, SEQ * BPW // LANES, remap_body, 0)

        def gather(s, b):
            return pltpu.make_async_copy(tab_hbm.at[idx_v.at[s]], rows[b], sems[b])

        for b in range(NBUF):
            gather(b, b).start()

        zero = jnp.zeros((LANES,), jnp.float32)

        def zero_body(r, _):
            for c in range(EMB // LANES):
                acc_v[r, pl.ds(c * LANES, LANES)] = zero
            return 0

        lax.fori_loop(0, BPW, zero_body, 0)

        def accum(buf):
            def body(i, _):
                r0 = i * UNROLL
                for u in range(UNROLL):
                    for c in range(EMB // LANES):
                        sl = pl.ds(c * LANES, LANES)
                        plsc.addupdate(acc_v.at[r0 + u, sl], buf[r0 + u, sl])
                return 0

            lax.fori_loop(0, BPW // UNROLL, body, 0)

        def outer(g, _):
            for b in range(NBUF):
                s = g * NBUF + b
                gather(s, b).wait()
                accum(rows[b])
                ns = s + NBUF

                @pl.when(ns < SEQ)
                def _():
                    gather(ns, b).start()

            return 0

        lax.fori_loop(0, SEQ // NBUF, outer, 0)

        pltpu.sync_copy(acc_v, out_hbm.at[pl.ds(base, BPW), :])

    return pool_kernel(x, table_lin)


def _tc_head(pooled, W, b2d):
    blk = 512

    def head_kernel(p_ref, w_ref, b_ref, o_ref):
        logits = jnp.dot(p_ref[...], w_ref[...],
                         preferred_element_type=jnp.float32) + b_ref[...]
        m = jnp.max(logits, axis=1, keepdims=True)
        z = logits - m
        lse = jnp.log(jnp.sum(jnp.exp(z), axis=1, keepdims=True))
        o_ref[...] = z - lse

    return pl.pallas_call(
        head_kernel,
        out_shape=jax.ShapeDtypeStruct((BATCH, OUT), jnp.float32),
        grid=(BATCH // blk,),
        in_specs=[
            pl.BlockSpec((blk, EMB), lambda i: (i, 0)),
            pl.BlockSpec((EMB, OUT), lambda i: (0, 0)),
            pl.BlockSpec((1, OUT), lambda i: (0, 0)),
        ],
        out_specs=pl.BlockSpec((blk, OUT), lambda i: (i, 0)),
    )(pooled, W, b2d)


def kernel(x, table, W, b):
    pairs = _tc_transpose(table.T)
    sums = _sc_pool(x, pairs.reshape(2 * SPLIT, EMB))
    return _tc_head(sums, W * jnp.float32(1.0 / SEQ), b.reshape(1, OUT))
